# Initial kernel scaffold; baseline (speedup 1.0000x reference)
import functools

import jax
import jax.numpy as jnp
from jax import lax
from jax.experimental import pallas as pl
from jax.experimental.pallas import tpu as pltpu
from jax.experimental.pallas import tpu_sc as plsc

ROWS = 100000
D = 128
NSEG = 512
NCLS = 10
NC = 2
NS = 16
NW = NC * NS
CHUNK = 200
HALF = 100
NCHUNKS = ROWS // CHUNK
ITERS = -(-NCHUNKS // NW)
CPAD = 16
SEG_T = NSEG // NS


def _sc_segment_sums(x, b3):
  mesh = plsc.VectorSubcoreMesh(core_axis_name="c", subcore_axis_name="s")

  @functools.partial(
      pl.kernel,
      out_type=(
          jax.ShapeDtypeStruct((NC, NSEG, D), jnp.float32),
          jax.ShapeDtypeStruct((NC, NSEG, CPAD), jnp.float32),
      ),
      mesh=mesh,
      scratch_types=dict(
          xbuf=pltpu.VMEM((CHUNK, D), jnp.float32),
          idx=pltpu.VMEM((2, HALF), jnp.int32),
          ones=pltpu.VMEM((HALF, CPAD), jnp.float32),
          zbuf=pltpu.VMEM((SEG_T, D), jnp.float32),
          zcbuf=pltpu.VMEM((SEG_T, CPAD), jnp.float32),
          obuf=pltpu.VMEM((SEG_T, D), jnp.float32),
          ocbuf=pltpu.VMEM((SEG_T, CPAD), jnp.float32),
          acc=pltpu.VMEM_SHARED((NSEG, D), jnp.float32),
          cacc=pltpu.VMEM_SHARED((NSEG, CPAD), jnp.float32),
      ),
  )
  def k(x_hbm, b_hbm, sums_hbm, cnt_hbm,
        xbuf, idx, ones, zbuf, zcbuf, obuf, ocbuf, acc, cacc):
    c = lax.axis_index("c")
    s = lax.axis_index("s")
    wid = s * NC + c

    zero = jnp.zeros((16,), jnp.float32)
    one = jnp.ones((16,), jnp.float32)
    for i in range(SEG_T):
      for j in range(D // 16):
        zbuf[i, pl.ds(j * 16, 16)] = zero
      zcbuf[i, :] = zero
    for i in range(HALF):
      ones[i, :] = one

    pltpu.sync_copy(zbuf, acc.at[pl.ds(s * SEG_T, SEG_T)])
    pltpu.sync_copy(zcbuf, cacc.at[pl.ds(s * SEG_T, SEG_T)])
    plsc.subcore_barrier()

    for kk in range(ITERS):
      chunk = wid + kk * NW

      @pl.when(chunk < NCHUNKS)
      def _():
        pltpu.sync_copy(x_hbm.at[pl.ds(chunk * CHUNK, CHUNK)], xbuf)
        pltpu.sync_copy(b_hbm.at[chunk], idx)
        for j in range(2):
          pltpu.sync_copy(xbuf.at[pl.ds(j * HALF, HALF)],
                          acc.at[idx.at[j]], add=True)
          pltpu.sync_copy(ones, cacc.at[idx.at[j]], add=True)

    plsc.subcore_barrier()
    pltpu.sync_copy(acc.at[pl.ds(s * SEG_T, SEG_T)], obuf)
    pltpu.sync_copy(obuf, sums_hbm.at[c, pl.ds(s * SEG_T, SEG_T)])
    pltpu.sync_copy(cacc.at[pl.ds(s * SEG_T, SEG_T)], ocbuf)
    pltpu.sync_copy(ocbuf, cnt_hbm.at[c, pl.ds(s * SEG_T, SEG_T)])

  return k(x, b3)


def _tc_finish(sums, cnts, w_pad):
  def body(s_ref, c_ref, w_ref, o_ref):
    sm = s_ref[0] + s_ref[1]
    cv = c_ref[0, :, 0:1] + c_ref[1, :, 0:1]
    h = sm / jnp.maximum(cv, 1.0)
    logits = lax.dot_general(h, w_ref[...], (((1,), (1,)), ((), ())),
                             preferred_element_type=jnp.float32)
    col = lax.broadcasted_iota(jnp.int32, (NSEG, 16), 1)
    valid = col < NCLS
    logits = jnp.where(valid, logits, jnp.float32(-1e30))
    m = jnp.max(logits, axis=1, keepdims=True)
    ex = jnp.where(valid, jnp.exp(logits - m), jnp.float32(0.0))
    lse = jnp.log(jnp.sum(ex, axis=1, keepdims=True)) + m
    o_ref[...] = logits - lse

  return pl.pallas_call(
      body,
      out_shape=jax.ShapeDtypeStruct((NSEG, 16), jnp.float32),
  )(sums, cnts, w_pad)


def kernel(x, batch, W):
  b3 = batch.astype(jnp.int32).reshape(NCHUNKS, 2, HALF)
  sums, cnts = _sc_segment_sums(x, b3)
  w_pad = jnp.zeros((16, D), jnp.float32).at[:NCLS].set(W)
  out = _tc_finish(sums, cnts, w_pad)
  return out[:, :NCLS]


# speedup vs baseline: 4.7469x; 4.7469x over previous
import functools

import jax
import jax.numpy as jnp
from jax import lax
from jax.experimental import pallas as pl
from jax.experimental.pallas import tpu as pltpu
from jax.experimental.pallas import tpu_sc as plsc

ROWS = 100000
D = 128
NSEG = 512
NCLS = 10
NC = 2
NS = 16
NW = NC * NS
CHUNK = 160
HALF = 80
NCHUNKS = ROWS // CHUNK
ITERS = -(-NCHUNKS // NW)
SEG_T = NSEG // NS
L = 16


def _sc_segment_sums(x, b3):
  mesh = plsc.VectorSubcoreMesh(core_axis_name="c", subcore_axis_name="s")

  @functools.partial(
      pl.kernel,
      out_type=(
          jax.ShapeDtypeStruct((NC, NSEG, D), jnp.float32),
          jax.ShapeDtypeStruct((NC, NSEG, D), jnp.float32),
      ),
      mesh=mesh,
      scratch_types=dict(
          xbuf=pltpu.VMEM((CHUNK, D), jnp.float32),
          idx=pltpu.VMEM((2, HALF), jnp.int32),
          ones=pltpu.VMEM((HALF, D), jnp.float32),
          zbuf=pltpu.VMEM((SEG_T, D), jnp.float32),
          obuf=pltpu.VMEM((SEG_T, D), jnp.float32),
          acc=pltpu.VMEM_SHARED((NSEG, D), jnp.float32),
          cacc=pltpu.VMEM_SHARED((NSEG, D), jnp.float32),
      ),
  )
  def k(x_hbm, b_hbm, sums_hbm, cnt_hbm, xbuf, idx, ones, zbuf, obuf, acc, cacc):
    c = lax.axis_index("c")
    s = lax.axis_index("s")
    wid = s * NC + c

    zero = jnp.zeros((L,), jnp.float32)
    one = jnp.ones((L,), jnp.float32)
    for i in range(SEG_T):
      for j in range(D // L):
        zbuf[i, pl.ds(j * L, L)] = zero
    for i in range(HALF):
      for j in range(D // L):
        ones[i, pl.ds(j * L, L)] = one

    pltpu.sync_copy(zbuf, acc.at[pl.ds(s * SEG_T, SEG_T)])
    pltpu.sync_copy(zbuf, cacc.at[pl.ds(s * SEG_T, SEG_T)])
    plsc.subcore_barrier()

    for kk in range(ITERS):
      chunk = wid + kk * NW

      @pl.when(chunk < NCHUNKS)
      def _():
        pltpu.sync_copy(x_hbm.at[pl.ds(chunk * CHUNK, CHUNK)], xbuf)
        pltpu.sync_copy(b_hbm.at[chunk], idx)
        for j in range(2):
          pltpu.sync_copy(xbuf.at[pl.ds(j * HALF, HALF)],
                          acc.at[idx.at[j]], add=True)
          pltpu.sync_copy(ones, cacc.at[idx.at[j]], add=True)

    plsc.subcore_barrier()
    pltpu.sync_copy(acc.at[pl.ds(s * SEG_T, SEG_T)], obuf)
    pltpu.sync_copy(obuf, sums_hbm.at[c, pl.ds(s * SEG_T, SEG_T)])
    pltpu.sync_copy(cacc.at[pl.ds(s * SEG_T, SEG_T)], obuf)
    pltpu.sync_copy(obuf, cnt_hbm.at[c, pl.ds(s * SEG_T, SEG_T)])

  return k(x, b3)


def _tc_finish(sums, cnts, w_pad):
  def body(s_ref, c_ref, w_ref, o_ref):
    sm = s_ref[0] + s_ref[1]
    cv = c_ref[0, :, 0:1] + c_ref[1, :, 0:1]
    h = sm / jnp.maximum(cv, 1.0)
    logits = lax.dot_general(h, w_ref[...], (((1,), (1,)), ((), ())),
                             preferred_element_type=jnp.float32)
    col = lax.broadcasted_iota(jnp.int32, (NSEG, 16), 1)
    valid = col < NCLS
    logits = jnp.where(valid, logits, jnp.float32(-1e30))
    m = jnp.max(logits, axis=1, keepdims=True)
    ex = jnp.where(valid, jnp.exp(logits - m), jnp.float32(0.0))
    lse = jnp.log(jnp.sum(ex, axis=1, keepdims=True)) + m
    o_ref[...] = logits - lse

  return pl.pallas_call(
      body,
      out_shape=jax.ShapeDtypeStruct((NSEG, 16), jnp.float32),
  )(sums, cnts, w_pad)


def kernel(x, batch, W):
  b3 = batch.astype(jnp.int32).reshape(NCHUNKS, 2, HALF)
  sums, cnts = _sc_segment_sums(x, b3)
  w_pad = jnp.zeros((16, D), jnp.float32).at[:NCLS].set(W)
  out = _tc_finish(sums, cnts, w_pad)
  return out[:, :NCLS]


# speedup vs baseline: 6.2007x; 1.3063x over previous
import functools

import jax
import jax.numpy as jnp
from jax import lax
from jax.experimental import pallas as pl
from jax.experimental.pallas import tpu as pltpu
from jax.experimental.pallas import tpu_sc as plsc

ROWS = 100000
D = 128
NSEG = 512
NCLS = 10
NC = 2
NS = 16
NW = NC * NS
CHUNK = 200
QUART = 100
NQ = CHUNK // QUART
NCHUNKS = ROWS // CHUNK
ITERS = -(-NCHUNKS // NW)
SEG_T = NSEG // NS
L = 16


def _sc_segment_sums(x, b3):
  mesh = plsc.VectorSubcoreMesh(core_axis_name="c", subcore_axis_name="s")

  @functools.partial(
      pl.kernel,
      out_type=(
          jax.ShapeDtypeStruct((NC, NSEG, D), jnp.float32),
          jax.ShapeDtypeStruct((NC, NSEG, D), jnp.float32),
      ),
      mesh=mesh,
      scratch_types=dict(
          xbuf0=pltpu.VMEM((CHUNK, D), jnp.float32),
          xbuf1=pltpu.VMEM((CHUNK, D), jnp.float32),
          idx=pltpu.VMEM((ITERS, NQ, QUART), jnp.int32),
          ones=pltpu.VMEM((QUART, D), jnp.float32),
          zbuf=pltpu.VMEM((SEG_T, D), jnp.float32),
          acc=pltpu.VMEM_SHARED((NSEG, D), jnp.float32),
          cacc=pltpu.VMEM_SHARED((NSEG, D), jnp.float32),
          sem0=pltpu.SemaphoreType.DMA,
          sem1=pltpu.SemaphoreType.DMA,
          isem=pltpu.SemaphoreType.DMA,
      ),
  )
  def k(x_hbm, b_hbm, sums_hbm, cnt_hbm,
        xbuf0, xbuf1, idx, ones, zbuf, acc, cacc, sem0, sem1, isem):
    c = lax.axis_index("c")
    s = lax.axis_index("s")
    wid = s * NC + c
    xbufs = [xbuf0, xbuf1]
    sems = [sem0, sem1]

    def chunk_of(kk):
      return wid + kk * NW

    for kk in range(ITERS):
      @pl.when(chunk_of(kk) < NCHUNKS)
      def _(kk=kk):
        pltpu.async_copy(b_hbm.at[chunk_of(kk)], idx.at[kk], isem)

    for kk in range(2):
      pltpu.async_copy(x_hbm.at[chunk_of(kk)], xbufs[kk], sems[kk])

    zero = jnp.zeros((L,), jnp.float32)
    one = jnp.ones((L,), jnp.float32)
    for i in range(SEG_T):
      for j in range(D // L):
        zbuf[i, pl.ds(j * L, L)] = zero
    for i in range(QUART):
      for j in range(D // L):
        ones[i, pl.ds(j * L, L)] = one

    pltpu.sync_copy(zbuf, acc.at[pl.ds(s * SEG_T, SEG_T)])
    pltpu.sync_copy(zbuf, cacc.at[pl.ds(s * SEG_T, SEG_T)])

    for kk in range(ITERS):
      @pl.when(chunk_of(kk) < NCHUNKS)
      def _(kk=kk):
        pltpu.make_async_copy(b_hbm.at[chunk_of(kk)], idx.at[kk], isem).wait()

    plsc.subcore_barrier()

    for kk in range(ITERS):
      p = kk % 2

      @pl.when(chunk_of(kk) < NCHUNKS)
      def _(kk=kk, p=p):
        pltpu.make_async_copy(x_hbm.at[chunk_of(kk)], xbufs[p],
                              sems[p]).wait()
        for j in range(NQ):
          pltpu.sync_copy(xbufs[p].at[pl.ds(j * QUART, QUART)],
                          acc.at[idx.at[kk, j]], add=True)
          pltpu.sync_copy(ones, cacc.at[idx.at[kk, j]], add=True)

      if kk + 2 < ITERS:
        @pl.when(chunk_of(kk + 2) < NCHUNKS)
        def _(kk=kk, p=p):
          pltpu.async_copy(x_hbm.at[chunk_of(kk + 2)], xbufs[p], sems[p])

    plsc.subcore_barrier()
    pltpu.sync_copy(acc.at[pl.ds(s * SEG_T, SEG_T)], zbuf)
    pltpu.sync_copy(zbuf, sums_hbm.at[c, pl.ds(s * SEG_T, SEG_T)])
    pltpu.sync_copy(cacc.at[pl.ds(s * SEG_T, SEG_T)], zbuf)
    pltpu.sync_copy(zbuf, cnt_hbm.at[c, pl.ds(s * SEG_T, SEG_T)])

  return k(x, b3)


def _tc_finish(sums, cnts, w_pad):
  def body(s_ref, c_ref, w_ref, o_ref):
    sm = s_ref[0] + s_ref[1]
    cv = c_ref[0, :, 0:1] + c_ref[1, :, 0:1]
    h = sm / jnp.maximum(cv, 1.0)
    logits = lax.dot_general(h, w_ref[...], (((1,), (1,)), ((), ())),
                             preferred_element_type=jnp.float32)
    col = lax.broadcasted_iota(jnp.int32, (NSEG, 16), 1)
    valid = col < NCLS
    logits = jnp.where(valid, logits, jnp.float32(-1e30))
    m = jnp.max(logits, axis=1, keepdims=True)
    ex = jnp.where(valid, jnp.exp(logits - m), jnp.float32(0.0))
    lse = jnp.log(jnp.sum(ex, axis=1, keepdims=True)) + m
    o_ref[...] = logits - lse

  return pl.pallas_call(
      body,
      out_shape=jax.ShapeDtypeStruct((NSEG, 16), jnp.float32),
  )(sums, cnts, w_pad)


def kernel(x, batch, W):
  b3 = batch.astype(jnp.int32).reshape(NCHUNKS, NQ, QUART)
  x3 = x.reshape(NCHUNKS, CHUNK, D)
  sums, cnts = _sc_segment_sums(x3, b3)
  w_pad = jnp.zeros((16, D), jnp.float32).at[:NCLS].set(W)
  out = _tc_finish(sums, cnts, w_pad)
  return out[:, :NCLS]


# speedup vs baseline: 6.2329x; 1.0052x over previous
import functools

import jax
import jax.numpy as jnp
from jax import lax
from jax.experimental import pallas as pl
from jax.experimental.pallas import tpu as pltpu
from jax.experimental.pallas import tpu_sc as plsc

ROWS = 100000
D = 128
NSEG = 512
NCLS = 10
NC = 2
NS = 16
NW = NC * NS
CHUNK = 200
QUART = 100
NQ = CHUNK // QUART
NCHUNKS = ROWS // CHUNK
ITERS = -(-NCHUNKS // NW)
SEG_T = NSEG // NS
L = 16


def _sc_segment_sums(x, b3):
  mesh = plsc.VectorSubcoreMesh(core_axis_name="c", subcore_axis_name="s")

  @functools.partial(
      pl.kernel,
      out_type=(
          jax.ShapeDtypeStruct((NC, NSEG, D), jnp.float32),
          jax.ShapeDtypeStruct((NC, NSEG, D), jnp.float32),
      ),
      mesh=mesh,
      scratch_types=dict(
          xbuf0=pltpu.VMEM((CHUNK, D), jnp.float32),
          xbuf1=pltpu.VMEM((CHUNK, D), jnp.float32),
          idx=pltpu.VMEM((ITERS, NQ, QUART), jnp.int32),
          ones=pltpu.VMEM((QUART, D), jnp.float32),
          zbuf=pltpu.VMEM((SEG_T, D), jnp.float32),
          acc=pltpu.VMEM_SHARED((NSEG, D), jnp.float32),
          cacc=pltpu.VMEM_SHARED((NSEG, D), jnp.float32),
          sem0=pltpu.SemaphoreType.DMA,
          sem1=pltpu.SemaphoreType.DMA,
          isem=pltpu.SemaphoreType.DMA,
          ssem=pltpu.SemaphoreType.DMA,
      ),
  )
  def k(x_hbm, b_hbm, sums_hbm, cnt_hbm,
        xbuf0, xbuf1, idx, ones, zbuf, acc, cacc, sem0, sem1, isem, ssem):
    c = lax.axis_index("c")
    s = lax.axis_index("s")
    wid = s * NC + c
    xbufs = [xbuf0, xbuf1]
    sems = [sem0, sem1]

    def chunk_of(kk):
      return wid + kk * NW

    for kk in range(ITERS):
      @pl.when(chunk_of(kk) < NCHUNKS)
      def _(kk=kk):
        pltpu.async_copy(b_hbm.at[chunk_of(kk)], idx.at[kk], isem)

    for kk in range(2):
      pltpu.async_copy(x_hbm.at[chunk_of(kk)], xbufs[kk], sems[kk])

    zero = jnp.zeros((L,), jnp.float32)
    one = jnp.ones((L,), jnp.float32)
    for i in range(SEG_T):
      for j in range(D // L):
        zbuf[i, pl.ds(j * L, L)] = zero
    for i in range(QUART):
      for j in range(D // L):
        ones[i, pl.ds(j * L, L)] = one

    pltpu.sync_copy(zbuf, acc.at[pl.ds(s * SEG_T, SEG_T)])
    pltpu.sync_copy(zbuf, cacc.at[pl.ds(s * SEG_T, SEG_T)])

    for kk in range(ITERS):
      @pl.when(chunk_of(kk) < NCHUNKS)
      def _(kk=kk):
        pltpu.make_async_copy(b_hbm.at[chunk_of(kk)], idx.at[kk], isem).wait()

    plsc.subcore_barrier()

    for kk in range(ITERS):
      p = kk % 2

      @pl.when(chunk_of(kk) < NCHUNKS)
      def _(kk=kk, p=p):
        pltpu.make_async_copy(x_hbm.at[chunk_of(kk)], xbufs[p],
                              sems[p]).wait()
        for j in range(NQ):
          pltpu.async_copy(xbufs[p].at[pl.ds(j * QUART, QUART)],
                           acc.at[idx.at[kk, j]], ssem, add=True)
          pltpu.async_copy(ones, cacc.at[idx.at[kk, j]], ssem, add=True)
        for j in range(NQ):
          pltpu.make_async_copy(xbufs[p].at[pl.ds(j * QUART, QUART)],
                                acc.at[idx.at[kk, j]], ssem).wait()
          pltpu.make_async_copy(ones, cacc.at[idx.at[kk, j]], ssem).wait()

      if kk + 2 < ITERS:
        @pl.when(chunk_of(kk + 2) < NCHUNKS)
        def _(kk=kk, p=p):
          pltpu.async_copy(x_hbm.at[chunk_of(kk + 2)], xbufs[p], sems[p])

    plsc.subcore_barrier()
    pltpu.sync_copy(acc.at[pl.ds(s * SEG_T, SEG_T)], zbuf)
    pltpu.sync_copy(zbuf, sums_hbm.at[c, pl.ds(s * SEG_T, SEG_T)])
    pltpu.sync_copy(cacc.at[pl.ds(s * SEG_T, SEG_T)], zbuf)
    pltpu.sync_copy(zbuf, cnt_hbm.at[c, pl.ds(s * SEG_T, SEG_T)])

  return k(x, b3)


def _tc_finish(sums, cnts, w_pad):
  def body(s_ref, c_ref, w_ref, o_ref):
    sm = s_ref[0] + s_ref[1]
    cv = c_ref[0, :, 0:1] + c_ref[1, :, 0:1]
    h = sm / jnp.maximum(cv, 1.0)
    logits = lax.dot_general(h, w_ref[...], (((1,), (1,)), ((), ())),
                             preferred_element_type=jnp.float32)
    col = lax.broadcasted_iota(jnp.int32, (NSEG, 16), 1)
    valid = col < NCLS
    logits = jnp.where(valid, logits, jnp.float32(-1e30))
    m = jnp.max(logits, axis=1, keepdims=True)
    ex = jnp.where(valid, jnp.exp(logits - m), jnp.float32(0.0))
    lse = jnp.log(jnp.sum(ex, axis=1, keepdims=True)) + m
    o_ref[...] = logits - lse

  return pl.pallas_call(
      body,
      out_shape=jax.ShapeDtypeStruct((NSEG, 16), jnp.float32),
  )(sums, cnts, w_pad)


def kernel(x, batch, W):
  b3 = batch.astype(jnp.int32).reshape(NCHUNKS, NQ, QUART)
  x3 = x.reshape(NCHUNKS, CHUNK, D)
  sums, cnts = _sc_segment_sums(x3, b3)
  w_pad = jnp.zeros((16, D), jnp.float32).at[:NCLS].set(W)
  out = _tc_finish(sums, cnts, w_pad)
  return out[:, :NCLS]


# speedup vs baseline: 8.4409x; 1.3542x over previous
import functools

import jax
import jax.numpy as jnp
from jax import lax
from jax.experimental import pallas as pl
from jax.experimental.pallas import tpu as pltpu
from jax.experimental.pallas import tpu_sc as plsc

ROWS = 100000
D = 128
NSEG = 512
NCLS = 10
NC = 2
NS = 16
NW = NC * NS
CHUNK = 200
QUART = 100
NQ = CHUNK // QUART
NCHUNKS = ROWS // CHUNK
ITERS = -(-NCHUNKS // NW)
SEG_T = NSEG // NS
L = 16
NB = 8
BJ = ROWS // NB


def _sc_segment_sums(x, b3):
  mesh = plsc.VectorSubcoreMesh(core_axis_name="c", subcore_axis_name="s")

  @functools.partial(
      pl.kernel,
      out_type=jax.ShapeDtypeStruct((NC, NSEG, D), jnp.float32),
      mesh=mesh,
      scratch_types=dict(
          xbuf0=pltpu.VMEM((CHUNK, D), jnp.float32),
          xbuf1=pltpu.VMEM((CHUNK, D), jnp.float32),
          idx=pltpu.VMEM((ITERS, NQ, QUART), jnp.int32),
          zbuf=pltpu.VMEM((SEG_T, D), jnp.float32),
          acc=pltpu.VMEM_SHARED((NSEG, D), jnp.float32),
          sem0=pltpu.SemaphoreType.DMA,
          sem1=pltpu.SemaphoreType.DMA,
          isem=pltpu.SemaphoreType.DMA,
          ssem=pltpu.SemaphoreType.DMA,
      ),
  )
  def k(x_hbm, b_hbm, sums_hbm,
        xbuf0, xbuf1, idx, zbuf, acc, sem0, sem1, isem, ssem):
    c = lax.axis_index("c")
    s = lax.axis_index("s")
    wid = s * NC + c
    xbufs = [xbuf0, xbuf1]
    sems = [sem0, sem1]

    def chunk_of(kk):
      return wid + kk * NW

    for kk in range(ITERS):
      @pl.when(chunk_of(kk) < NCHUNKS)
      def _(kk=kk):
        pltpu.async_copy(b_hbm.at[chunk_of(kk)], idx.at[kk], isem)

    for kk in range(2):
      pltpu.async_copy(x_hbm.at[chunk_of(kk)], xbufs[kk], sems[kk])

    zero = jnp.zeros((L,), jnp.float32)
    for i in range(SEG_T):
      for j in range(D // L):
        zbuf[i, pl.ds(j * L, L)] = zero

    pltpu.sync_copy(zbuf, acc.at[pl.ds(s * SEG_T, SEG_T)])

    for kk in range(ITERS):
      @pl.when(chunk_of(kk) < NCHUNKS)
      def _(kk=kk):
        pltpu.make_async_copy(b_hbm.at[chunk_of(kk)], idx.at[kk], isem).wait()

    plsc.subcore_barrier()

    for kk in range(ITERS):
      p = kk % 2

      @pl.when(chunk_of(kk) < NCHUNKS)
      def _(kk=kk, p=p):
        pltpu.make_async_copy(x_hbm.at[chunk_of(kk)], xbufs[p],
                              sems[p]).wait()
        for j in range(NQ):
          pltpu.async_copy(xbufs[p].at[pl.ds(j * QUART, QUART)],
                           acc.at[idx.at[kk, j]], ssem, add=True)
        for j in range(NQ):
          pltpu.make_async_copy(xbufs[p].at[pl.ds(j * QUART, QUART)],
                                acc.at[idx.at[kk, j]], ssem).wait()

      if kk + 2 < ITERS:
        @pl.when(chunk_of(kk + 2) < NCHUNKS)
        def _(kk=kk, p=p):
          pltpu.async_copy(x_hbm.at[chunk_of(kk + 2)], xbufs[p], sems[p])

    plsc.subcore_barrier()
    pltpu.sync_copy(acc.at[pl.ds(s * SEG_T, SEG_T)], zbuf)
    pltpu.sync_copy(zbuf, sums_hbm.at[c, pl.ds(s * SEG_T, SEG_T)])

  return k(x, b3)


def _tc_finish(sums, b2, w_pad):
  def body(s_ref, b_ref, w_ref, o_ref):
    cm = jnp.zeros((16, 32), jnp.float32)
    for j in range(NB):
      ids = b_ref[j]
      hi = (ids >> 5)[None, :]
      lo = (ids & 31)[None, :]
      hh = (lax.broadcasted_iota(jnp.int32, (16, BJ), 0) == hi
            ).astype(jnp.float32)
      ll = (lax.broadcasted_iota(jnp.int32, (32, BJ), 0) == lo
            ).astype(jnp.float32)
      cm = cm + lax.dot_general(hh, ll, (((1,), (1,)), ((), ())),
                                preferred_element_type=jnp.float32)
    srow = lax.broadcasted_iota(jnp.int32, (NSEG, 16), 1)
    sidx = lax.broadcasted_iota(jnp.int32, (NSEG, 16), 0)
    hsel = (srow == (sidx >> 5)).astype(jnp.float32)
    t = lax.dot_general(hsel, cm, (((1,), (0,)), ((), ())),
                        preferred_element_type=jnp.float32)
    scol = lax.broadcasted_iota(jnp.int32, (NSEG, 32), 1)
    sidx2 = lax.broadcasted_iota(jnp.int32, (NSEG, 32), 0)
    cv = jnp.sum(jnp.where(scol == (sidx2 & 31), t, 0.0),
                 axis=1, keepdims=True)

    sm = s_ref[0] + s_ref[1]
    h = sm / jnp.maximum(cv, 1.0)
    logits = lax.dot_general(h, w_ref[...], (((1,), (1,)), ((), ())),
                             preferred_element_type=jnp.float32)
    col = lax.broadcasted_iota(jnp.int32, (NSEG, 16), 1)
    valid = col < NCLS
    logits = jnp.where(valid, logits, jnp.float32(-1e30))
    m = jnp.max(logits, axis=1, keepdims=True)
    ex = jnp.where(valid, jnp.exp(logits - m), jnp.float32(0.0))
    lse = jnp.log(jnp.sum(ex, axis=1, keepdims=True)) + m
    o_ref[...] = logits - lse

  return pl.pallas_call(
      body,
      out_shape=jax.ShapeDtypeStruct((NSEG, 16), jnp.float32),
  )(sums, b2, w_pad)


def kernel(x, batch, W):
  batch = batch.astype(jnp.int32)
  b3 = batch.reshape(NCHUNKS, NQ, QUART)
  b2 = batch.reshape(NB, BJ)
  x3 = x.reshape(NCHUNKS, CHUNK, D)
  sums = _sc_segment_sums(x3, b3)
  w_pad = jnp.zeros((16, D), jnp.float32).at[:NCLS].set(W)
  out = _tc_finish(sums, b2, w_pad)
  return out[:, :NCLS]


# speedup vs baseline: 8.5554x; 1.0136x over previous
import functools

import jax
import jax.numpy as jnp
from jax import lax
from jax.experimental import pallas as pl
from jax.experimental.pallas import tpu as pltpu
from jax.experimental.pallas import tpu_sc as plsc

ROWS = 100000
D = 128
NSEG = 512
NCLS = 10
NC = 2
NS = 16
NW = NC * NS
CHUNK = 200
QUART = 100
NQ = CHUNK // QUART
NCHUNKS = ROWS // CHUNK
ITERS = -(-NCHUNKS // NW)
SEG_T = NSEG // NS
L = 16
NB = 8
BJ = ROWS // NB


def _sc_segment_sums(x, b3):
  mesh = plsc.VectorSubcoreMesh(core_axis_name="c", subcore_axis_name="s")

  @functools.partial(
      pl.kernel,
      out_type=jax.ShapeDtypeStruct((NC, NSEG, D), jnp.float32),
      mesh=mesh,
      scratch_types=dict(
          xbuf0=pltpu.VMEM((CHUNK, D), jnp.float32),
          xbuf1=pltpu.VMEM((CHUNK, D), jnp.float32),
          xbuf2=pltpu.VMEM((CHUNK, D), jnp.float32),
          idx=pltpu.VMEM((ITERS, NQ, QUART), jnp.int32),
          zbuf=pltpu.VMEM((SEG_T, D), jnp.float32),
          acc=pltpu.VMEM_SHARED((NSEG, D), jnp.float32),
          sem0=pltpu.SemaphoreType.DMA,
          sem1=pltpu.SemaphoreType.DMA,
          sem2=pltpu.SemaphoreType.DMA,
          isem=pltpu.SemaphoreType.DMA,
          ssem=pltpu.SemaphoreType.DMA,
      ),
  )
  def k(x_hbm, b_hbm, sums_hbm,
        xbuf0, xbuf1, xbuf2, idx, zbuf, acc, sem0, sem1, sem2, isem, ssem):
    c = lax.axis_index("c")
    s = lax.axis_index("s")
    wid = s * NC + c
    xbufs = [xbuf0, xbuf1, xbuf2]
    sems = [sem0, sem1, sem2]
    NBUF = 3

    def chunk_of(kk):
      return wid + kk * NW

    for kk in range(ITERS):
      @pl.when(chunk_of(kk) < NCHUNKS)
      def _(kk=kk):
        pltpu.async_copy(b_hbm.at[chunk_of(kk)], idx.at[kk], isem)

    for kk in range(NBUF):
      pltpu.async_copy(x_hbm.at[chunk_of(kk)], xbufs[kk], sems[kk])

    zero = jnp.zeros((L,), jnp.float32)
    for i in range(SEG_T):
      for j in range(D // L):
        zbuf[i, pl.ds(j * L, L)] = zero

    pltpu.sync_copy(zbuf, acc.at[pl.ds(s * SEG_T, SEG_T)])

    for kk in range(ITERS):
      @pl.when(chunk_of(kk) < NCHUNKS)
      def _(kk=kk):
        pltpu.make_async_copy(b_hbm.at[chunk_of(kk)], idx.at[kk], isem).wait()

    plsc.subcore_barrier()

    for kk in range(ITERS):
      p = kk % NBUF

      @pl.when(chunk_of(kk) < NCHUNKS)
      def _(kk=kk, p=p):
        pltpu.make_async_copy(x_hbm.at[chunk_of(kk)], xbufs[p],
                              sems[p]).wait()
        for j in range(NQ):
          pltpu.async_copy(xbufs[p].at[pl.ds(j * QUART, QUART)],
                           acc.at[idx.at[kk, j]], ssem, add=True)
        for j in range(NQ):
          pltpu.make_async_copy(xbufs[p].at[pl.ds(j * QUART, QUART)],
                                acc.at[idx.at[kk, j]], ssem).wait()

      if kk + NBUF < ITERS:
        @pl.when(chunk_of(kk + NBUF) < NCHUNKS)
        def _(kk=kk, p=p):
          pltpu.async_copy(x_hbm.at[chunk_of(kk + NBUF)], xbufs[p], sems[p])

    plsc.subcore_barrier()
    pltpu.sync_copy(acc.at[pl.ds(s * SEG_T, SEG_T)], zbuf)
    pltpu.sync_copy(zbuf, sums_hbm.at[c, pl.ds(s * SEG_T, SEG_T)])

  return k(x, b3)


def _tc_finish(sums, b2, w_pad):
  def body(s_ref, b_ref, w_ref, o_ref):
    cm = jnp.zeros((16, 32), jnp.float32)
    for j in range(NB):
      ids = b_ref[j]
      hi = (ids >> 5)[None, :]
      lo = (ids & 31)[None, :]
      hh = (lax.broadcasted_iota(jnp.int32, (16, BJ), 0) == hi
            ).astype(jnp.float32)
      ll = (lax.broadcasted_iota(jnp.int32, (32, BJ), 0) == lo
            ).astype(jnp.float32)
      cm = cm + lax.dot_general(hh, ll, (((1,), (1,)), ((), ())),
                                preferred_element_type=jnp.float32)
    srow = lax.broadcasted_iota(jnp.int32, (NSEG, 16), 1)
    sidx = lax.broadcasted_iota(jnp.int32, (NSEG, 16), 0)
    hsel = (srow == (sidx >> 5)).astype(jnp.float32)
    t = lax.dot_general(hsel, cm, (((1,), (0,)), ((), ())),
                        preferred_element_type=jnp.float32)
    scol = lax.broadcasted_iota(jnp.int32, (NSEG, 32), 1)
    sidx2 = lax.broadcasted_iota(jnp.int32, (NSEG, 32), 0)
    cv = jnp.sum(jnp.where(scol == (sidx2 & 31), t, 0.0),
                 axis=1, keepdims=True)

    sm = s_ref[0] + s_ref[1]
    h = sm / jnp.maximum(cv, 1.0)
    w = jnp.concatenate(
        [w_ref[...], jnp.zeros((16 - NCLS, D), jnp.float32)], axis=0)
    logits = lax.dot_general(h, w, (((1,), (1,)), ((), ())),
                             preferred_element_type=jnp.float32)
    col = lax.broadcasted_iota(jnp.int32, (NSEG, 16), 1)
    valid = col < NCLS
    logits = jnp.where(valid, logits, jnp.float32(-1e30))
    m = jnp.max(logits, axis=1, keepdims=True)
    ex = jnp.where(valid, jnp.exp(logits - m), jnp.float32(0.0))
    lse = jnp.log(jnp.sum(ex, axis=1, keepdims=True)) + m
    o_ref[...] = (logits - lse)[:, :NCLS]

  return pl.pallas_call(
      body,
      out_shape=jax.ShapeDtypeStruct((NSEG, NCLS), jnp.float32),
  )(sums, b2, w_pad)


def kernel(x, batch, W):
  batch = batch.astype(jnp.int32)
  b3 = batch.reshape(NCHUNKS, NQ, QUART)
  b2 = batch.reshape(NB, BJ)
  x3 = x.reshape(NCHUNKS, CHUNK, D)
  sums = _sc_segment_sums(x3, b3)
  return _tc_finish(sums, b2, W)


# speedup vs baseline: 8.8268x; 1.0317x over previous
import functools

import jax
import jax.numpy as jnp
from jax import lax
from jax.experimental import pallas as pl
from jax.experimental.pallas import tpu as pltpu
from jax.experimental.pallas import tpu_sc as plsc

ROWS = 100000
D = 128
NSEG = 512
NCLS = 10
NC = 2
NS = 16
NW = NC * NS
CHUNK = 200
QUART = 100
NQ = CHUNK // QUART
NCHUNKS = ROWS // CHUNK
ITERS = -(-NCHUNKS // NW)
SEG_T = NSEG // NS
L = 16
NB = 8
BJ = ROWS // NB


def _sc_segment_sums(x, b3):
  mesh = plsc.VectorSubcoreMesh(core_axis_name="c", subcore_axis_name="s")

  @functools.partial(
      pl.kernel,
      out_type=jax.ShapeDtypeStruct((NC, NSEG, D), jnp.float32),
      mesh=mesh,
      scratch_types=dict(
          xbuf0=pltpu.VMEM((CHUNK, D), jnp.float32),
          xbuf1=pltpu.VMEM((CHUNK, D), jnp.float32),
          xbuf2=pltpu.VMEM((CHUNK, D), jnp.float32),
          idx=pltpu.VMEM((ITERS, NQ, QUART), jnp.int32),
          zbuf=pltpu.VMEM((SEG_T, D), jnp.float32),
          acc=pltpu.VMEM_SHARED((NSEG, D), jnp.float32),
          sem0=pltpu.SemaphoreType.DMA,
          sem1=pltpu.SemaphoreType.DMA,
          sem2=pltpu.SemaphoreType.DMA,
          isem=pltpu.SemaphoreType.DMA,
          ssem=pltpu.SemaphoreType.DMA,
      ),
  )
  def k(x_hbm, b_hbm, sums_hbm,
        xbuf0, xbuf1, xbuf2, idx, zbuf, acc, sem0, sem1, sem2, isem, ssem):
    c = lax.axis_index("c")
    s = lax.axis_index("s")
    wid = s * NC + c
    xbufs = [xbuf0, xbuf1, xbuf2]
    sems = [sem0, sem1, sem2]
    NBUF = 3

    def chunk_of(kk):
      return wid + kk * NW

    for kk in range(ITERS):
      @pl.when(chunk_of(kk) < NCHUNKS)
      def _(kk=kk):
        pltpu.async_copy(b_hbm.at[chunk_of(kk)], idx.at[kk], isem)

    for kk in range(NBUF):
      pltpu.async_copy(x_hbm.at[chunk_of(kk)], xbufs[kk], sems[kk])

    zero = jnp.zeros((L,), jnp.float32)
    for i in range(SEG_T):
      for j in range(D // L):
        zbuf[i, pl.ds(j * L, L)] = zero

    pltpu.sync_copy(zbuf, acc.at[pl.ds(s * SEG_T, SEG_T)])

    for kk in range(ITERS):
      @pl.when(chunk_of(kk) < NCHUNKS)
      def _(kk=kk):
        pltpu.make_async_copy(b_hbm.at[chunk_of(kk)], idx.at[kk], isem).wait()

    plsc.subcore_barrier()

    for kk in range(ITERS):
      p = kk % NBUF

      @pl.when(chunk_of(kk) < NCHUNKS)
      def _(kk=kk, p=p):
        pltpu.make_async_copy(x_hbm.at[chunk_of(kk)], xbufs[p],
                              sems[p]).wait()
        for j in range(NQ):
          pltpu.async_copy(xbufs[p].at[pl.ds(j * QUART, QUART)],
                           acc.at[idx.at[kk, j]], ssem, add=True)
        for j in range(NQ):
          pltpu.make_async_copy(xbufs[p].at[pl.ds(j * QUART, QUART)],
                                acc.at[idx.at[kk, j]], ssem).wait()

      if kk + NBUF < ITERS:
        @pl.when(chunk_of(kk + NBUF) < NCHUNKS)
        def _(kk=kk, p=p):
          pltpu.async_copy(x_hbm.at[chunk_of(kk + NBUF)], xbufs[p], sems[p])

    plsc.subcore_barrier()
    pltpu.sync_copy(acc.at[pl.ds(s * SEG_T, SEG_T)], zbuf)
    pltpu.sync_copy(zbuf, sums_hbm.at[c, pl.ds(s * SEG_T, SEG_T)])

  return k(x, b3)


def _tc_counts(b2):
  def body(b_ref, o_ref):
    cm = jnp.zeros((16, 32), jnp.float32)
    for j in range(NB):
      ids = b_ref[j]
      hi = (ids >> 5)[None, :]
      lo = (ids & 31)[None, :]
      hh = (lax.broadcasted_iota(jnp.int32, (16, BJ), 0) == hi
            ).astype(jnp.bfloat16)
      ll = (lax.broadcasted_iota(jnp.int32, (32, BJ), 0) == lo
            ).astype(jnp.bfloat16)
      cm = cm + lax.dot_general(hh, ll, (((1,), (1,)), ((), ())),
                                preferred_element_type=jnp.float32)
    srow = lax.broadcasted_iota(jnp.int32, (NSEG, 16), 1)
    sidx = lax.broadcasted_iota(jnp.int32, (NSEG, 16), 0)
    hsel = (srow == (sidx >> 5)).astype(jnp.float32)
    t = lax.dot_general(hsel, cm, (((1,), (0,)), ((), ())),
                        preferred_element_type=jnp.float32)
    scol = lax.broadcasted_iota(jnp.int32, (NSEG, 32), 1)
    sidx2 = lax.broadcasted_iota(jnp.int32, (NSEG, 32), 0)
    o_ref[...] = jnp.sum(jnp.where(scol == (sidx2 & 31), t, 0.0),
                         axis=1, keepdims=True)

  return pl.pallas_call(
      body,
      out_shape=jax.ShapeDtypeStruct((NSEG, 1), jnp.float32),
  )(b2)


def _tc_finish(sums, cv, w_pad):
  def body(s_ref, c_ref, w_ref, o_ref):
    cv = c_ref[...]
    sm = s_ref[0] + s_ref[1]
    h = sm / jnp.maximum(cv, 1.0)
    w = jnp.concatenate(
        [w_ref[...], jnp.zeros((16 - NCLS, D), jnp.float32)], axis=0)
    logits = lax.dot_general(h, w, (((1,), (1,)), ((), ())),
                             preferred_element_type=jnp.float32)
    col = lax.broadcasted_iota(jnp.int32, (NSEG, 16), 1)
    valid = col < NCLS
    logits = jnp.where(valid, logits, jnp.float32(-1e30))
    m = jnp.max(logits, axis=1, keepdims=True)
    ex = jnp.where(valid, jnp.exp(logits - m), jnp.float32(0.0))
    lse = jnp.log(jnp.sum(ex, axis=1, keepdims=True)) + m
    o_ref[...] = (logits - lse)[:, :NCLS]

  return pl.pallas_call(
      body,
      out_shape=jax.ShapeDtypeStruct((NSEG, NCLS), jnp.float32),
  )(sums, cv, w_pad)


def kernel(x, batch, W):
  batch = batch.astype(jnp.int32)
  b3 = batch.reshape(NCHUNKS, NQ, QUART)
  b2 = batch.reshape(NB, BJ)
  x3 = x.reshape(NCHUNKS, CHUNK, D)
  cv = _tc_counts(b2)
  sums = _sc_segment_sums(x3, b3)
  return _tc_finish(sums, cv, W)


# speedup vs baseline: 9.3002x; 1.0536x over previous
import functools

import jax
import jax.numpy as jnp
from jax import lax
from jax.experimental import pallas as pl
from jax.experimental.pallas import tpu as pltpu
from jax.experimental.pallas import tpu_sc as plsc

ROWS = 100000
D = 128
NSEG = 512
NCLS = 10
NC = 2
NS = 16
NW = NC * NS
CHUNK = 200
QUART = 100
NQ = CHUNK // QUART
NCHUNKS = ROWS // CHUNK
SC_CHUNKS = 300
ITERS = -(-SC_CHUNKS // NW)
SEG_T = NSEG // NS
L = 16
BJ = 2000
NB = ROWS // BJ
TC_B0 = (SC_CHUNKS * CHUNK) // BJ
TC_NB = NB - TC_B0


def _sc_segment_sums(x3, b3):
  mesh = plsc.VectorSubcoreMesh(core_axis_name="c", subcore_axis_name="s")

  @functools.partial(
      pl.kernel,
      out_type=jax.ShapeDtypeStruct((NC, NSEG, D), jnp.float32),
      mesh=mesh,
      scratch_types=dict(
          xbuf0=pltpu.VMEM((CHUNK, D), jnp.float32),
          xbuf1=pltpu.VMEM((CHUNK, D), jnp.float32),
          xbuf2=pltpu.VMEM((CHUNK, D), jnp.float32),
          idx=pltpu.VMEM((ITERS, NQ, QUART), jnp.int32),
          zbuf=pltpu.VMEM((SEG_T, D), jnp.float32),
          acc=pltpu.VMEM_SHARED((NSEG, D), jnp.float32),
          sem0=pltpu.SemaphoreType.DMA,
          sem1=pltpu.SemaphoreType.DMA,
          sem2=pltpu.SemaphoreType.DMA,
          isem=pltpu.SemaphoreType.DMA,
          ssem=pltpu.SemaphoreType.DMA,
      ),
  )
  def k(x_hbm, b_hbm, sums_hbm,
        xbuf0, xbuf1, xbuf2, idx, zbuf, acc, sem0, sem1, sem2, isem, ssem):
    c = lax.axis_index("c")
    s = lax.axis_index("s")
    wid = s * NC + c
    xbufs = [xbuf0, xbuf1, xbuf2]
    sems = [sem0, sem1, sem2]
    NBUF = 3

    def chunk_of(kk):
      return wid + kk * NW

    for kk in range(ITERS):
      @pl.when(chunk_of(kk) < SC_CHUNKS)
      def _(kk=kk):
        pltpu.async_copy(b_hbm.at[chunk_of(kk)], idx.at[kk], isem)

    for kk in range(NBUF):
      pltpu.async_copy(x_hbm.at[chunk_of(kk)], xbufs[kk], sems[kk])

    zero = jnp.zeros((L,), jnp.float32)
    for i in range(SEG_T):
      for j in range(D // L):
        zbuf[i, pl.ds(j * L, L)] = zero

    pltpu.sync_copy(zbuf, acc.at[pl.ds(s * SEG_T, SEG_T)])

    for kk in range(ITERS):
      @pl.when(chunk_of(kk) < SC_CHUNKS)
      def _(kk=kk):
        pltpu.make_async_copy(b_hbm.at[chunk_of(kk)], idx.at[kk], isem).wait()

    plsc.subcore_barrier()

    for kk in range(ITERS):
      p = kk % NBUF

      @pl.when(chunk_of(kk) < SC_CHUNKS)
      def _(kk=kk, p=p):
        pltpu.make_async_copy(x_hbm.at[chunk_of(kk)], xbufs[p],
                              sems[p]).wait()
        for j in range(NQ):
          pltpu.async_copy(xbufs[p].at[pl.ds(j * QUART, QUART)],
                           acc.at[idx.at[kk, j]], ssem, add=True)
        for j in range(NQ):
          pltpu.make_async_copy(xbufs[p].at[pl.ds(j * QUART, QUART)],
                                acc.at[idx.at[kk, j]], ssem).wait()

      if kk + NBUF < ITERS:
        @pl.when(chunk_of(kk + NBUF) < SC_CHUNKS)
        def _(kk=kk, p=p):
          pltpu.async_copy(x_hbm.at[chunk_of(kk + NBUF)], xbufs[p], sems[p])

    plsc.subcore_barrier()
    pltpu.sync_copy(acc.at[pl.ds(s * SEG_T, SEG_T)], zbuf)
    pltpu.sync_copy(zbuf, sums_hbm.at[c, pl.ds(s * SEG_T, SEG_T)])

  return k(x3, b3)


def _tc_counts(b2):
  def body(b_ref, o_ref):
    cm = jnp.zeros((16, 32), jnp.float32)
    for j in range(NB):
      ids = b_ref[j, 0]
      hi = (ids >> 5)[None, :]
      lo = (ids & 31)[None, :]
      hh = (lax.broadcasted_iota(jnp.int32, (16, BJ), 0) == hi
            ).astype(jnp.bfloat16)
      ll = (lax.broadcasted_iota(jnp.int32, (32, BJ), 0) == lo
            ).astype(jnp.bfloat16)
      cm = cm + lax.dot_general(hh, ll, (((1,), (1,)), ((), ())),
                                preferred_element_type=jnp.float32)
    srow = lax.broadcasted_iota(jnp.int32, (NSEG, 16), 1)
    sidx = lax.broadcasted_iota(jnp.int32, (NSEG, 16), 0)
    hsel = (srow == (sidx >> 5)).astype(jnp.float32)
    t = lax.dot_general(hsel, cm, (((1,), (0,)), ((), ())),
                        preferred_element_type=jnp.float32)
    scol = lax.broadcasted_iota(jnp.int32, (NSEG, 32), 1)
    sidx2 = lax.broadcasted_iota(jnp.int32, (NSEG, 32), 0)
    o_ref[...] = jnp.sum(jnp.where(scol == (sidx2 & 31), t, 0.0),
                         axis=1, keepdims=True)

  return pl.pallas_call(
      body,
      out_shape=jax.ShapeDtypeStruct((NSEG, 1), jnp.float32),
  )(b2)


def _tc_segment_sums(x, b2):
  def body(b_ref, x_ref, o_ref):
    j = pl.program_id(0)
    ids = b_ref[0, 0][None, :]
    oh = (lax.broadcasted_iota(jnp.int32, (NSEG, BJ), 0) == ids
          ).astype(jnp.float32)
    contrib = lax.dot_general(oh, x_ref[...], (((1,), (0,)), ((), ())),
                              preferred_element_type=jnp.float32)

    @pl.when(j == 0)
    def _():
      o_ref[...] = contrib

    @pl.when(j > 0)
    def _():
      o_ref[...] = o_ref[...] + contrib

  return pl.pallas_call(
      body,
      grid=(TC_NB,),
      in_specs=[
          pl.BlockSpec((1, 1, BJ), lambda j: (TC_B0 + j, 0, 0)),
          pl.BlockSpec((BJ, D), lambda j: (TC_B0 + j, 0)),
      ],
      out_specs=pl.BlockSpec((NSEG, D), lambda j: (0, 0)),
      out_shape=jax.ShapeDtypeStruct((NSEG, D), jnp.float32),
  )(b2, x)


def _tc_finish(sums_sc, sums_tc, cv, w):
  def body(s_ref, t_ref, c_ref, w_ref, o_ref):
    sm = s_ref[0] + s_ref[1] + t_ref[...]
    h = sm / jnp.maximum(c_ref[...], 1.0)
    wp = jnp.concatenate(
        [w_ref[...], jnp.zeros((16 - NCLS, D), jnp.float32)], axis=0)
    logits = lax.dot_general(h, wp, (((1,), (1,)), ((), ())),
                             preferred_element_type=jnp.float32)
    col = lax.broadcasted_iota(jnp.int32, (NSEG, 16), 1)
    valid = col < NCLS
    logits = jnp.where(valid, logits, jnp.float32(-1e30))
    m = jnp.max(logits, axis=1, keepdims=True)
    ex = jnp.where(valid, jnp.exp(logits - m), jnp.float32(0.0))
    lse = jnp.log(jnp.sum(ex, axis=1, keepdims=True)) + m
    o_ref[...] = (logits - lse)[:, :NCLS]

  return pl.pallas_call(
      body,
      out_shape=jax.ShapeDtypeStruct((NSEG, NCLS), jnp.float32),
  )(sums_sc, sums_tc, cv, w)


def kernel(x, batch, W):
  batch = batch.astype(jnp.int32)
  b3 = batch.reshape(NCHUNKS, NQ, QUART)
  b2 = batch.reshape(NB, 1, BJ)
  x3 = x.reshape(NCHUNKS, CHUNK, D)
  cv = _tc_counts(b2)
  sums_tc = _tc_segment_sums(x, b2)
  sums_sc = _sc_segment_sums(x3, b3)
  return _tc_finish(sums_sc, sums_tc, cv, W)
